# Initial kernel scaffold; baseline (speedup 1.0000x reference)
import jax
import jax.numpy as jnp
from jax.experimental import pallas as pl


def kernel(pred, label):
    raise NotImplementedError("write your pallas kernel here")


# speedup vs baseline: 210.0813x; 210.0813x over previous
import functools

import jax
import jax.numpy as jnp
from jax import lax
from jax.experimental import pallas as pl
from jax.experimental.pallas import tpu as pltpu
from jax.experimental.pallas import tpu_sc as plsc

NCLS = 21
CPAD = 32
LANES = 16
BATCH = 8
HW = 512 * 512
TOTAL = BATCH * HW
NCORES = 2
NSUB = 16
NWORK = NCORES * NSUB
PER_W = TOTAL // NWORK
CHUNK = 32768
NCHUNK = PER_W // CHUNK
HISTW = LANES * CPAD


def _dice_body(pred_hbm, label_hbm, out_hbm,
               pbuf, lbuf, hp, hl, hm, tmp, obuf, shared, shared2,
               sem_p, sem_l):
    c = lax.axis_index("c")
    s = lax.axis_index("s")
    w = c * NSUB + s
    zeros16 = jnp.zeros((LANES,), jnp.float32)
    ones16 = jnp.ones((LANES,), jnp.float32)
    lane = lax.iota(jnp.int32, LANES) * CPAD

    for i in range(CPAD):
        hp[pl.ds(i * LANES, LANES)] = zeros16
        hl[pl.ds(i * LANES, LANES)] = zeros16
        hm[pl.ds(i * LANES, LANES)] = zeros16

    base = w * PER_W
    for ch in range(NCHUNK):
        off = base + ch * CHUNK
        cp_p = pltpu.make_async_copy(pred_hbm.at[pl.ds(off, CHUNK)], pbuf, sem_p)
        cp_l = pltpu.make_async_copy(label_hbm.at[pl.ds(off, CHUNK)], lbuf, sem_l)
        cp_p.start()
        cp_l.start()
        cp_p.wait()
        cp_l.wait()

        def body(i, carry):
            p16 = pbuf[pl.ds(i * LANES, LANES)]
            l16 = lbuf[pl.ds(i * LANES, LANES)]
            ip = lane + p16
            il = lane + l16
            plsc.addupdate_scatter(hp, [ip], ones16)
            plsc.addupdate_scatter(hl, [il], ones16)
            plsc.addupdate_scatter(hm, [ip], ones16, mask=p16 == l16)
            return carry

        lax.fori_loop(0, CHUNK // LANES, body, 0)

    pltpu.sync_copy(hp, shared.at[pl.ds((s * 3 + 0) * HISTW, HISTW)])
    pltpu.sync_copy(hl, shared.at[pl.ds((s * 3 + 1) * HISTW, HISTW)])
    pltpu.sync_copy(hm, shared.at[pl.ds((s * 3 + 2) * HISTW, HISTW)])
    plsc.subcore_barrier()

    @pl.when(s < 4)
    def _():
        accs = []
        for h in range(3):
            a0 = zeros16
            a1 = zeros16
            for q in range(4):
                pltpu.sync_copy(
                    shared.at[pl.ds(((s * 4 + q) * 3 + h) * HISTW, HISTW)], tmp)
                for ln in range(LANES):
                    a0 = a0 + tmp[pl.ds(ln * CPAD, LANES)]
                    a1 = a1 + tmp[pl.ds(ln * CPAD + LANES, LANES)]
            accs.append((a0, a1))
        (p0, p1), (l0, l1), (m0, m1) = accs
        eps = jnp.float32(1e-10)
        inv_b = jnp.float32(1.0 / BATCH)
        s0 = (2.0 * m0) / (p0 + l0 + eps) * inv_b
        s1 = (2.0 * m1) / (p1 + l1 + eps) * inv_b
        obuf[pl.ds(0, LANES)] = s0
        obuf[pl.ds(LANES, LANES)] = s1
        pltpu.sync_copy(obuf, shared2.at[pl.ds(s * CPAD, CPAD)])

    plsc.subcore_barrier()

    @pl.when(s == 0)
    def _():
        t0 = zeros16
        t1 = zeros16
        for q in range(4):
            pltpu.sync_copy(shared2.at[pl.ds(q * CPAD, CPAD)], obuf)
            t0 = t0 + obuf[pl.ds(0, LANES)]
            t1 = t1 + obuf[pl.ds(LANES, LANES)]
        obuf[pl.ds(0, LANES)] = t0
        obuf[pl.ds(LANES, LANES)] = t1
        pltpu.sync_copy(obuf, out_hbm.at[pl.ds(c * CPAD, CPAD)])


@jax.jit
def _dice_call(pred_flat, label_flat):
    mesh = plsc.VectorSubcoreMesh(
        core_axis_name="c", subcore_axis_name="s",
        num_cores=NCORES, num_subcores=NSUB)
    return pl.kernel(
        _dice_body,
        out_type=jax.ShapeDtypeStruct((NCORES * CPAD,), jnp.float32),
        mesh=mesh,
        compiler_params=pltpu.CompilerParams(needs_layout_passes=False),
        scratch_types=[
            pltpu.VMEM((CHUNK,), jnp.int32),
            pltpu.VMEM((CHUNK,), jnp.int32),
            pltpu.VMEM((HISTW,), jnp.float32),
            pltpu.VMEM((HISTW,), jnp.float32),
            pltpu.VMEM((HISTW,), jnp.float32),
            pltpu.VMEM((HISTW,), jnp.float32),
            pltpu.VMEM((CPAD,), jnp.float32),
            pltpu.VMEM_SHARED((NSUB * 3 * HISTW,), jnp.float32),
            pltpu.VMEM_SHARED((4 * CPAD,), jnp.float32),
            pltpu.SemaphoreType.DMA,
            pltpu.SemaphoreType.DMA,
        ],
    )(pred_flat, label_flat)


def kernel(pred, label):
    parts = _dice_call(pred.reshape(TOTAL), label.reshape(TOTAL))
    return (parts[:NCLS] + parts[CPAD:CPAD + NCLS])


# speedup vs baseline: 222.0499x; 1.0570x over previous
import functools

import jax
import jax.numpy as jnp
from jax import lax
from jax.experimental import pallas as pl
from jax.experimental.pallas import tpu as pltpu
from jax.experimental.pallas import tpu_sc as plsc

NCLS = 21
CPAD = 32
LANES = 16
BATCH = 8
HW = 512 * 512
TOTAL = BATCH * HW
NCORES = 2
NSUB = 16
NWORK = NCORES * NSUB
PER_W = TOTAL // NWORK
CHUNK = 16384
NCHUNK = PER_W // CHUNK
HISTW = LANES * CPAD


def _dice_body(pred_hbm, label_hbm, out_hbm,
               pbuf0, lbuf0, pbuf1, lbuf1, hp, hl, hm, tmp, obuf,
               shared, shared2, sem_p, sem_l):
    c = lax.axis_index("c")
    s = lax.axis_index("s")
    w = c * NSUB + s
    zeros16 = jnp.zeros((LANES,), jnp.float32)
    ones16 = jnp.ones((LANES,), jnp.float32)
    lane = lax.iota(jnp.int32, LANES) * CPAD

    base = w * PER_W
    bufs = [(pbuf0, lbuf0), (pbuf1, lbuf1)]

    def start_chunk(ch):
        off = base + ch * CHUNK
        pb, lb = bufs[ch % 2]
        cp = pltpu.make_async_copy(pred_hbm.at[pl.ds(off, CHUNK)], pb, sem_p)
        cl = pltpu.make_async_copy(label_hbm.at[pl.ds(off, CHUNK)], lb, sem_l)
        cp.start()
        cl.start()
        return cp, cl

    pending = start_chunk(0)

    for i in range(CPAD):
        hp[pl.ds(i * LANES, LANES)] = zeros16
        hl[pl.ds(i * LANES, LANES)] = zeros16
        hm[pl.ds(i * LANES, LANES)] = zeros16

    for ch in range(NCHUNK):
        pending[0].wait()
        pending[1].wait()
        if ch + 1 < NCHUNK:
            pending = start_chunk(ch + 1)
        pb, lb = bufs[ch % 2]

        def body(i, carry):
            p16 = pb[pl.ds(i * LANES, LANES)]
            l16 = lb[pl.ds(i * LANES, LANES)]
            ip = lane + p16
            il = lane + l16
            plsc.addupdate_scatter(hp, [ip], ones16)
            plsc.addupdate_scatter(hl, [il], ones16)
            plsc.addupdate_scatter(hm, [ip], ones16, mask=p16 == l16)
            return carry

        lax.fori_loop(0, CHUNK // LANES, body, 0)

    pltpu.sync_copy(hp, shared.at[pl.ds((s * 3 + 0) * HISTW, HISTW)])
    pltpu.sync_copy(hl, shared.at[pl.ds((s * 3 + 1) * HISTW, HISTW)])
    pltpu.sync_copy(hm, shared.at[pl.ds((s * 3 + 2) * HISTW, HISTW)])
    plsc.subcore_barrier()

    @pl.when(s < 4)
    def _():
        accs = []
        for h in range(3):
            a0 = zeros16
            a1 = zeros16
            for q in range(4):
                pltpu.sync_copy(
                    shared.at[pl.ds(((s * 4 + q) * 3 + h) * HISTW, HISTW)], tmp)
                for ln in range(LANES):
                    a0 = a0 + tmp[pl.ds(ln * CPAD, LANES)]
                    a1 = a1 + tmp[pl.ds(ln * CPAD + LANES, LANES)]
            accs.append((a0, a1))
        (p0, p1), (l0, l1), (m0, m1) = accs
        eps = jnp.float32(1e-10)
        inv_b = jnp.float32(1.0 / BATCH)
        s0 = (2.0 * m0) / (p0 + l0 + eps) * inv_b
        s1 = (2.0 * m1) / (p1 + l1 + eps) * inv_b
        obuf[pl.ds(0, LANES)] = s0
        obuf[pl.ds(LANES, LANES)] = s1
        pltpu.sync_copy(obuf, shared2.at[pl.ds(s * CPAD, CPAD)])

    plsc.subcore_barrier()

    @pl.when(s == 0)
    def _():
        t0 = zeros16
        t1 = zeros16
        for q in range(4):
            pltpu.sync_copy(shared2.at[pl.ds(q * CPAD, CPAD)], obuf)
            t0 = t0 + obuf[pl.ds(0, LANES)]
            t1 = t1 + obuf[pl.ds(LANES, LANES)]
        obuf[pl.ds(0, LANES)] = t0
        obuf[pl.ds(LANES, LANES)] = t1
        pltpu.sync_copy(obuf, out_hbm.at[pl.ds(c * CPAD, CPAD)])


@jax.jit
def _dice_call(pred_flat, label_flat):
    mesh = plsc.VectorSubcoreMesh(
        core_axis_name="c", subcore_axis_name="s",
        num_cores=NCORES, num_subcores=NSUB)
    return pl.kernel(
        _dice_body,
        out_type=jax.ShapeDtypeStruct((NCORES * CPAD,), jnp.float32),
        mesh=mesh,
        compiler_params=pltpu.CompilerParams(needs_layout_passes=False),
        scratch_types=[
            pltpu.VMEM((CHUNK,), jnp.int32),
            pltpu.VMEM((CHUNK,), jnp.int32),
            pltpu.VMEM((CHUNK,), jnp.int32),
            pltpu.VMEM((CHUNK,), jnp.int32),
            pltpu.VMEM((HISTW,), jnp.float32),
            pltpu.VMEM((HISTW,), jnp.float32),
            pltpu.VMEM((HISTW,), jnp.float32),
            pltpu.VMEM((HISTW,), jnp.float32),
            pltpu.VMEM((CPAD,), jnp.float32),
            pltpu.VMEM_SHARED((NSUB * 3 * HISTW,), jnp.float32),
            pltpu.VMEM_SHARED((4 * CPAD,), jnp.float32),
            pltpu.SemaphoreType.DMA,
            pltpu.SemaphoreType.DMA,
        ],
    )(pred_flat, label_flat)


def kernel(pred, label):
    parts = _dice_call(pred.reshape(TOTAL), label.reshape(TOTAL))
    return (parts[:NCLS] + parts[CPAD:CPAD + NCLS])
